# baseline (speedup 1.0000x reference)
import functools

import jax
import jax.numpy as jnp
from jax import lax
from jax.experimental import pallas as pl
from jax.experimental.pallas import tpu as pltpu
from jax.experimental.pallas import tpu_sc as plsc

N, E, D = 10000, 320000, 128
PAD = 136
NC, NS = 2, 16
NW = NC * NS
EPW = E // NW
K = 40
NB = EPW // K
RPT = N // NS
WR = 25
LN = 16


def _proj_body(x_ref, wi_ref, wj_ref, pd_ref, ps_ref):
    xb = x_ref[:, :]
    pd_ref[:, :] = jnp.dot(xb, wi_ref[:, :], preferred_element_type=jnp.float32)
    ps_ref[:, :] = jnp.dot(xb, wj_ref[:, :], preferred_element_type=jnp.float32)


def _edge_affine_body(ea_ref, we_ref, b1_ref, a_ref):
    ea = ea_ref[:, :]
    a = jnp.dot(ea, we_ref[:, :], preferred_element_type=jnp.float32) + b1_ref[:, :]
    a_ref[:, :] = a.astype(jnp.bfloat16)


def _final_body(hs_ref, x_ref, w2_ref, b2_ref, g1x_ref, g1a_ref, gb1_ref,
                gw2_ref, gb2_ref, o_ref):
    hs = hs_ref[0] + hs_ref[1]
    h = hs[:, :D]
    c = hs[:, D:D + 1]
    hm = h / jnp.maximum(c, 1.0)
    pos = (c > 0.0).astype(jnp.float32)
    aggr = jnp.dot(hm, w2_ref[:, :], preferred_element_type=jnp.float32) \
        + b2_ref[:, :] * pos
    xb = x_ref[:, :]
    u = jnp.dot(xb, g1x_ref[:, :], preferred_element_type=jnp.float32) \
        + jnp.dot(aggr, g1a_ref[:, :], preferred_element_type=jnp.float32) \
        + gb1_ref[:, :]
    u = jnp.where(u > 0.0, u, jnp.exp(jnp.minimum(u, 0.0)) - 1.0)
    o_ref[:, :] = jnp.dot(u, gw2_ref[:, :], preferred_element_type=jnp.float32) \
        + gb2_ref[:, :]


_sc_mesh = plsc.VectorSubcoreMesh(core_axis_name="c", subcore_axis_name="s")


@functools.partial(
    pl.kernel,
    out_type=jax.ShapeDtypeStruct((NC, N, PAD), jnp.float32),
    mesh=_sc_mesh,
    scratch_types=[
        pltpu.VMEM((2, 2, K), jnp.int32),
        pltpu.VMEM((2, K), jnp.int32),
        pltpu.VMEM((2, K, D), jnp.float32),
        pltpu.VMEM((2, K, D), jnp.float32),
        pltpu.VMEM((2, K, D), jnp.bfloat16),
        pltpu.VMEM((2, K, PAD), jnp.float32),
        pltpu.VMEM_SHARED((N, PAD), jnp.float32),
        pltpu.SemaphoreType.DMA,
        pltpu.SemaphoreType.DMA,
        pltpu.SemaphoreType.DMA,
        pltpu.SemaphoreType.DMA,
        pltpu.SemaphoreType.DMA,
        pltpu.SemaphoreType.DMA,
        pltpu.SemaphoreType.DMA,
        pltpu.SemaphoreType.DMA,
    ],
    compiler_params=pltpu.CompilerParams(use_tc_tiling_on_sc=False,
                                         needs_layout_passes=False),
)
def _sc_accumulate(pd_hbm, ps_hbm, a_hbm, ei_hbm, dst_hbm, z_hbm, hinit_hbm,
                   out_hbm, gidx, sidx, gd, gs, ab, hb, acc,
                   gi0, gi1, di0, di1, sg0, sg1, ss0, ss1):
    cid = lax.axis_index("c")
    tid = lax.axis_index("s")
    wid = cid * NS + tid
    b0 = wid * NB
    sem_i, sem_d = (gi0, gi1), (di0, di1)
    sem_g, sem_s = (sg0, sg1), (ss0, ss1)

    pltpu.sync_copy(hinit_hbm, hb.at[0])
    pltpu.sync_copy(hinit_hbm, hb.at[1])

    pltpu.sync_copy(z_hbm, acc.at[pl.ds(tid * RPT, RPT)])
    plsc.subcore_barrier()

    def issue_gathers(b, s):
        pltpu.async_copy(a_hbm.at[pl.ds((b0 + b) * K, K)], ab.at[s], sem_g[s])
        pltpu.async_copy(pd_hbm.at[gidx.at[s, 0]], gd.at[s], sem_g[s])
        pltpu.async_copy(ps_hbm.at[gidx.at[s, 1]], gs.at[s], sem_g[s])

    def wait_gathers(s):
        pltpu.make_async_copy(a_hbm.at[pl.ds(0, K)], ab.at[s], sem_g[s]).wait()
        pltpu.make_async_copy(pd_hbm.at[gidx.at[s, 0]], gd.at[s], sem_g[s]).wait()
        pltpu.make_async_copy(ps_hbm.at[gidx.at[s, 1]], gs.at[s], sem_g[s]).wait()

    pltpu.sync_copy(ei_hbm.at[pl.ds(b0, 1)], gidx.at[pl.ds(0, 1)])
    issue_gathers(0, 0)
    pltpu.async_copy(dst_hbm.at[pl.ds(b0, 1)], sidx.at[pl.ds(0, 1)], sem_d[0])
    pltpu.async_copy(ei_hbm.at[pl.ds(b0 + 1, 1)], gidx.at[pl.ds(1, 1)],
                     sem_i[1])

    @pl.loop(0, NB // 2)
    def _pair(g):
        for s in range(2):
            b = 2 * g + s
            ns = 1 - s

            @pl.when(b + 1 < NB)
            def _():
                @pl.when(b >= 1)
                def _():
                    pltpu.make_async_copy(hb.at[ns], acc.at[sidx.at[ns]],
                                          sem_s[ns]).wait()
                pltpu.async_copy(dst_hbm.at[pl.ds(b0 + b + 1, 1)],
                                 sidx.at[pl.ds(ns, 1)], sem_d[ns])
                pltpu.make_async_copy(ei_hbm.at[pl.ds(0, 1)],
                                      gidx.at[pl.ds(ns, 1)], sem_i[ns]).wait()
                issue_gathers(b + 1, ns)

            wait_gathers(s)

            @pl.when(b + 2 < NB)
            def _():
                pltpu.async_copy(ei_hbm.at[pl.ds(b0 + b + 2, 1)],
                                 gidx.at[pl.ds(s, 1)], sem_i[s])

            @plsc.parallel_loop(0, K, unroll=4)
            def _edge(e):
                for p in range(D // (2 * LN)):
                    a_lo, a_hi = plsc.unpack(ab[s, e, pl.ds(2 * LN * p, 2 * LN)],
                                             format=plsc.PackFormat.INTERLEAVED)
                    for j, av in ((2 * p, a_lo), (2 * p + 1, a_hi)):
                        sl = pl.ds(j * LN, LN)
                        t = gd[s, e, sl] + gs[s, e, sl] + av
                        neg = jnp.exp(t) - 1.0
                        hb[s, e, sl] = jnp.where(t > 0.0, t, neg)

            pltpu.make_async_copy(dst_hbm.at[pl.ds(0, 1)],
                                  sidx.at[pl.ds(s, 1)], sem_d[s]).wait()
            pltpu.async_copy(hb.at[s], acc.at[sidx.at[s]], sem_s[s], add=True)

    for s in range(2):
        pltpu.make_async_copy(hb.at[s], acc.at[sidx.at[s]], sem_s[s]).wait()

    plsc.subcore_barrier()
    row0 = tid * RPT
    pltpu.sync_copy(acc.at[pl.ds(row0, RPT)], out_hbm.at[cid, pl.ds(row0, RPT)])


def kernel(x, edge_index, edge_attr, phi_W1, phi_b1, phi_W2, phi_b2,
           gam_W1, gam_b1, gam_W2, gam_b2):
    w1i, w1j, w1e = phi_W1[:D], phi_W1[D:2 * D], phi_W1[2 * D:]
    g1x, g1a = gam_W1[:D], gam_W1[D:]
    perm = jnp.arange(D).reshape(D // (2 * LN), 2, LN).transpose(0, 2, 1).reshape(D)
    w1e = w1e[:, perm]
    b1_p = phi_b1[perm]
    src2d = edge_index[0].reshape(E // K, K)
    dst2d = edge_index[1].reshape(E // K, K)
    ei2 = jnp.stack([dst2d, src2d], axis=1)
    zeros = jnp.zeros((RPT, PAD), jnp.float32)
    hinit = jnp.zeros((K, PAD), jnp.float32).at[:, D].set(1.0)

    pd, ps = pl.pallas_call(
        _proj_body,
        grid=(10,),
        in_specs=[pl.BlockSpec((N // 10, D), lambda i: (i, 0)),
                  pl.BlockSpec((D, D), lambda i: (0, 0)),
                  pl.BlockSpec((D, D), lambda i: (0, 0))],
        out_specs=[pl.BlockSpec((N // 10, D), lambda i: (i, 0))] * 2,
        out_shape=[jax.ShapeDtypeStruct((N, D), jnp.float32)] * 2,
    )(x, w1i, w1j)

    BA = 4000
    abf = pl.pallas_call(
        _edge_affine_body,
        grid=(E // BA,),
        in_specs=[pl.BlockSpec((BA, 4), lambda i: (i, 0)),
                  pl.BlockSpec((4, D), lambda i: (0, 0)),
                  pl.BlockSpec((1, D), lambda i: (0, 0))],
        out_specs=pl.BlockSpec((BA, D), lambda i: (i, 0)),
        out_shape=jax.ShapeDtypeStruct((E, D), jnp.bfloat16),
    )(edge_attr, w1e, b1_p.reshape(1, D))

    hs2 = _sc_accumulate(pd, ps, abf, ei2, dst2d, zeros, hinit)

    BF = 1000
    out = pl.pallas_call(
        _final_body,
        grid=(N // BF,),
        in_specs=[pl.BlockSpec((NC, BF, PAD), lambda i: (0, i, 0)),
                  pl.BlockSpec((BF, D), lambda i: (i, 0)),
                  pl.BlockSpec((D, D), lambda i: (0, 0)),
                  pl.BlockSpec((1, D), lambda i: (0, 0)),
                  pl.BlockSpec((D, D), lambda i: (0, 0)),
                  pl.BlockSpec((D, D), lambda i: (0, 0)),
                  pl.BlockSpec((1, D), lambda i: (0, 0)),
                  pl.BlockSpec((D, D), lambda i: (0, 0)),
                  pl.BlockSpec((1, D), lambda i: (0, 0))],
        out_specs=pl.BlockSpec((BF, D), lambda i: (i, 0)),
        out_shape=jax.ShapeDtypeStruct((N, D), jnp.float32),
    )(hs2, x, phi_W2, phi_b2.reshape(1, D), g1x, g1a,
      gam_b1.reshape(1, D), gam_W2, gam_b2.reshape(1, D))

    return out


# speedup vs baseline: 1.3737x; 1.3737x over previous
import functools

import jax
import jax.numpy as jnp
from jax import lax
from jax.experimental import pallas as pl
from jax.experimental.pallas import tpu as pltpu
from jax.experimental.pallas import tpu_sc as plsc

N, E, D = 10000, 320000, 128
PAD = 136
NC, NS = 2, 16
NW = NC * NS
EPW = E // NW
K = 40
NB = EPW // K
RPT = N // NS
WR = 25
LN = 16


def _proj_body(x_ref, wi_ref, wj_ref, pd_ref, ps_ref):
    xb = x_ref[:, :]
    pd_ref[:, :] = jnp.dot(xb, wi_ref[:, :], preferred_element_type=jnp.float32)
    ps_ref[:, :] = jnp.dot(xb, wj_ref[:, :], preferred_element_type=jnp.float32)


def _edge_affine_body(ea_ref, we_ref, b1_ref, a_ref):
    ea = ea_ref[:, :]
    a_ref[:, :] = jnp.dot(ea, we_ref[:, :],
                          preferred_element_type=jnp.float32) + b1_ref[:, :]


def _final_body(hs_ref, x_ref, w2_ref, b2_ref, g1x_ref, g1a_ref, gb1_ref,
                gw2_ref, gb2_ref, o_ref):
    hs = hs_ref[0] + hs_ref[1]
    h = hs[:, :D]
    c = hs[:, D:D + 1]
    hm = h / jnp.maximum(c, 1.0)
    pos = (c > 0.0).astype(jnp.float32)
    aggr = jnp.dot(hm, w2_ref[:, :], preferred_element_type=jnp.float32) \
        + b2_ref[:, :] * pos
    xb = x_ref[:, :]
    u = jnp.dot(xb, g1x_ref[:, :], preferred_element_type=jnp.float32) \
        + jnp.dot(aggr, g1a_ref[:, :], preferred_element_type=jnp.float32) \
        + gb1_ref[:, :]
    u = jnp.where(u > 0.0, u, jnp.exp(jnp.minimum(u, 0.0)) - 1.0)
    o_ref[:, :] = jnp.dot(u, gw2_ref[:, :], preferred_element_type=jnp.float32) \
        + gb2_ref[:, :]


_sc_mesh = plsc.VectorSubcoreMesh(core_axis_name="c", subcore_axis_name="s")


@functools.partial(
    pl.kernel,
    out_type=jax.ShapeDtypeStruct((NC, N, PAD), jnp.float32),
    mesh=_sc_mesh,
    scratch_types=[
        pltpu.VMEM((2, 2, K), jnp.int32),
        pltpu.VMEM((2, K), jnp.int32),
        pltpu.VMEM((2, K, D), jnp.float32),
        pltpu.VMEM((2, K, D), jnp.float32),
        pltpu.VMEM((2, K, D), jnp.float32),
        pltpu.VMEM((2, K, PAD), jnp.float32),
        pltpu.VMEM_SHARED((N, PAD), jnp.float32),
        pltpu.SemaphoreType.DMA,
        pltpu.SemaphoreType.DMA,
        pltpu.SemaphoreType.DMA,
        pltpu.SemaphoreType.DMA,
        pltpu.SemaphoreType.DMA,
        pltpu.SemaphoreType.DMA,
        pltpu.SemaphoreType.DMA,
        pltpu.SemaphoreType.DMA,
    ],
    compiler_params=pltpu.CompilerParams(use_tc_tiling_on_sc=False),
)
def _sc_accumulate(pd_hbm, ps_hbm, a_hbm, ei_hbm, dst_hbm, z_hbm, hinit_hbm,
                   out_hbm, gidx, sidx, gd, gs, ab, hb, acc,
                   gi0, gi1, di0, di1, sg0, sg1, ss0, ss1):
    cid = lax.axis_index("c")
    tid = lax.axis_index("s")
    wid = cid * NS + tid
    b0 = wid * NB
    sem_i, sem_d = (gi0, gi1), (di0, di1)
    sem_g, sem_s = (sg0, sg1), (ss0, ss1)

    pltpu.sync_copy(hinit_hbm, hb.at[0])
    pltpu.sync_copy(hinit_hbm, hb.at[1])

    pltpu.sync_copy(z_hbm, acc.at[pl.ds(tid * RPT, RPT)])
    plsc.subcore_barrier()

    def issue_gathers(b, s):
        pltpu.async_copy(a_hbm.at[pl.ds((b0 + b) * K, K)], ab.at[s], sem_g[s])
        pltpu.async_copy(pd_hbm.at[gidx.at[s, 0]], gd.at[s], sem_g[s])
        pltpu.async_copy(ps_hbm.at[gidx.at[s, 1]], gs.at[s], sem_g[s])

    def wait_gathers(s):
        pltpu.make_async_copy(a_hbm.at[pl.ds(0, K)], ab.at[s], sem_g[s]).wait()
        pltpu.make_async_copy(pd_hbm.at[gidx.at[s, 0]], gd.at[s], sem_g[s]).wait()
        pltpu.make_async_copy(ps_hbm.at[gidx.at[s, 1]], gs.at[s], sem_g[s]).wait()

    pltpu.sync_copy(ei_hbm.at[pl.ds(b0, 1)], gidx.at[pl.ds(0, 1)])
    issue_gathers(0, 0)
    pltpu.async_copy(dst_hbm.at[pl.ds(b0, 1)], sidx.at[pl.ds(0, 1)], sem_d[0])
    pltpu.async_copy(ei_hbm.at[pl.ds(b0 + 1, 1)], gidx.at[pl.ds(1, 1)],
                     sem_i[1])

    @pl.loop(0, NB // 2)
    def _pair(g):
        for s in range(2):
            b = 2 * g + s
            ns = 1 - s

            @pl.when(b + 1 < NB)
            def _():
                pltpu.make_async_copy(ei_hbm.at[pl.ds(0, 1)],
                                      gidx.at[pl.ds(ns, 1)], sem_i[ns]).wait()
                issue_gathers(b + 1, ns)

                @pl.when(b >= 1)
                def _():
                    pltpu.make_async_copy(hb.at[ns], acc.at[sidx.at[ns]],
                                          sem_s[ns]).wait()
                pltpu.async_copy(dst_hbm.at[pl.ds(b0 + b + 1, 1)],
                                 sidx.at[pl.ds(ns, 1)], sem_d[ns])

            wait_gathers(s)

            @pl.when(b + 2 < NB)
            def _():
                pltpu.async_copy(ei_hbm.at[pl.ds(b0 + b + 2, 1)],
                                 gidx.at[pl.ds(s, 1)], sem_i[s])

            @plsc.parallel_loop(0, K, unroll=4)
            def _edge(e):
                for j in range(D // LN):
                    sl = pl.ds(j * LN, LN)
                    t = gd[s, e, sl] + gs[s, e, sl] + ab[s, e, sl]
                    neg = jnp.exp(t) - 1.0
                    hb[s, e, sl] = jnp.where(t > 0.0, t, neg)

            pltpu.make_async_copy(dst_hbm.at[pl.ds(0, 1)],
                                  sidx.at[pl.ds(s, 1)], sem_d[s]).wait()
            pltpu.async_copy(hb.at[s], acc.at[sidx.at[s]], sem_s[s], add=True)

    for s in range(2):
        pltpu.make_async_copy(hb.at[s], acc.at[sidx.at[s]], sem_s[s]).wait()

    plsc.subcore_barrier()
    row0 = tid * RPT
    pltpu.sync_copy(acc.at[pl.ds(row0, RPT)], out_hbm.at[cid, pl.ds(row0, RPT)])


def kernel(x, edge_index, edge_attr, phi_W1, phi_b1, phi_W2, phi_b2,
           gam_W1, gam_b1, gam_W2, gam_b2):
    w1i, w1j, w1e = phi_W1[:D], phi_W1[D:2 * D], phi_W1[2 * D:]
    g1x, g1a = gam_W1[:D], gam_W1[D:]
    src2d = edge_index[0].reshape(E // K, K)
    dst2d = edge_index[1].reshape(E // K, K)
    ei2 = jnp.stack([dst2d, src2d], axis=1)
    zeros = jnp.zeros((RPT, PAD), jnp.float32)
    hinit = jnp.zeros((K, PAD), jnp.float32).at[:, D].set(1.0)

    pd, ps = pl.pallas_call(
        _proj_body,
        grid=(10,),
        in_specs=[pl.BlockSpec((N // 10, D), lambda i: (i, 0)),
                  pl.BlockSpec((D, D), lambda i: (0, 0)),
                  pl.BlockSpec((D, D), lambda i: (0, 0))],
        out_specs=[pl.BlockSpec((N // 10, D), lambda i: (i, 0))] * 2,
        out_shape=[jax.ShapeDtypeStruct((N, D), jnp.float32)] * 2,
    )(x, w1i, w1j)

    BA = 4000
    abf = pl.pallas_call(
        _edge_affine_body,
        grid=(E // BA,),
        in_specs=[pl.BlockSpec((BA, 4), lambda i: (i, 0)),
                  pl.BlockSpec((4, D), lambda i: (0, 0)),
                  pl.BlockSpec((1, D), lambda i: (0, 0))],
        out_specs=pl.BlockSpec((BA, D), lambda i: (i, 0)),
        out_shape=jax.ShapeDtypeStruct((E, D), jnp.float32),
    )(edge_attr, w1e, phi_b1.reshape(1, D))

    hs2 = _sc_accumulate(pd, ps, abf, ei2, dst2d, zeros, hinit)

    BF = 1000
    out = pl.pallas_call(
        _final_body,
        grid=(N // BF,),
        in_specs=[pl.BlockSpec((NC, BF, PAD), lambda i: (0, i, 0)),
                  pl.BlockSpec((BF, D), lambda i: (i, 0)),
                  pl.BlockSpec((D, D), lambda i: (0, 0)),
                  pl.BlockSpec((1, D), lambda i: (0, 0)),
                  pl.BlockSpec((D, D), lambda i: (0, 0)),
                  pl.BlockSpec((D, D), lambda i: (0, 0)),
                  pl.BlockSpec((1, D), lambda i: (0, 0)),
                  pl.BlockSpec((D, D), lambda i: (0, 0)),
                  pl.BlockSpec((1, D), lambda i: (0, 0))],
        out_specs=pl.BlockSpec((BF, D), lambda i: (i, 0)),
        out_shape=jax.ShapeDtypeStruct((N, D), jnp.float32),
    )(hs2, x, phi_W2, phi_b2.reshape(1, D), g1x, g1a,
      gam_b1.reshape(1, D), gam_W2, gam_b2.reshape(1, D))

    return out


# speedup vs baseline: 1.3738x; 1.0000x over previous
import functools

import jax
import jax.numpy as jnp
from jax import lax
from jax.experimental import pallas as pl
from jax.experimental.pallas import tpu as pltpu
from jax.experimental.pallas import tpu_sc as plsc

N, E, D = 10000, 320000, 128
PAD = 136
NC, NS = 2, 16
NW = NC * NS
EPW = E // NW
K = 40
NB = EPW // K
RPT = N // NS
LN = 16


def _proj_body(x_ref, wi_ref, wj_ref, pd_ref, ps_ref):
    xb = x_ref[:, :]
    pd_ref[:, :] = jnp.dot(xb, wi_ref[:, :], preferred_element_type=jnp.float32)
    ps_ref[:, :] = jnp.dot(xb, wj_ref[:, :], preferred_element_type=jnp.float32)


def _edge_affine_body(ea_ref, we_ref, b1_ref, a_ref):
    ea = ea_ref[:, :]
    a_ref[:, :] = jnp.dot(ea, we_ref[:, :],
                          preferred_element_type=jnp.float32) + b1_ref[:, :]


def _final_body(hs_ref, x_ref, w2_ref, b2_ref, g1x_ref, g1a_ref, gb1_ref,
                gw2_ref, gb2_ref, o_ref):
    hs = hs_ref[0] + hs_ref[1]
    h = hs[:, :D]
    c = hs[:, D:D + 1]
    hm = h / jnp.maximum(c, 1.0)
    pos = (c > 0.0).astype(jnp.float32)
    aggr = jnp.dot(hm, w2_ref[:, :], preferred_element_type=jnp.float32) \
        + b2_ref[:, :] * pos
    xb = x_ref[:, :]
    u = jnp.dot(xb, g1x_ref[:, :], preferred_element_type=jnp.float32) \
        + jnp.dot(aggr, g1a_ref[:, :], preferred_element_type=jnp.float32) \
        + gb1_ref[:, :]
    u = jnp.where(u > 0.0, u, jnp.exp(jnp.minimum(u, 0.0)) - 1.0)
    o_ref[:, :] = jnp.dot(u, gw2_ref[:, :], preferred_element_type=jnp.float32) \
        + gb2_ref[:, :]


_sc_mesh = plsc.VectorSubcoreMesh(core_axis_name="c", subcore_axis_name="s")


@functools.partial(
    pl.kernel,
    out_type=jax.ShapeDtypeStruct((NC, N, PAD), jnp.float32),
    mesh=_sc_mesh,
    scratch_types=[
        pltpu.VMEM((2, 2, K), jnp.int32),
        pltpu.VMEM((2, K), jnp.int32),
        pltpu.VMEM((2, K, D), jnp.float32),
        pltpu.VMEM((2, K, D), jnp.float32),
        pltpu.VMEM((2, K, D), jnp.float32),
        pltpu.VMEM((2, K, PAD), jnp.float32),
        pltpu.VMEM_SHARED((N, PAD), jnp.float32),
        pltpu.SemaphoreType.DMA,
        pltpu.SemaphoreType.DMA,
        pltpu.SemaphoreType.DMA,
        pltpu.SemaphoreType.DMA,
        pltpu.SemaphoreType.DMA,
        pltpu.SemaphoreType.DMA,
        pltpu.SemaphoreType.DMA,
        pltpu.SemaphoreType.DMA,
    ],
    compiler_params=pltpu.CompilerParams(use_tc_tiling_on_sc=False),
)
def _sc_accumulate(pd_hbm, ps_hbm, a_hbm, ei_hbm, dst_hbm, z_hbm, hinit_hbm,
                   out_hbm, gidx, sidx, gd, gs, ab, hb, acc,
                   gi0, gi1, di0, di1, sg0, sg1, ss0, ss1):
    cid = lax.axis_index("c")
    tid = lax.axis_index("s")
    wid = cid * NS + tid
    b0 = wid * NB
    sem_i, sem_d = (gi0, gi1), (di0, di1)
    sem_g, sem_s = (sg0, sg1), (ss0, ss1)

    pltpu.sync_copy(hinit_hbm, hb.at[0])
    pltpu.sync_copy(hinit_hbm, hb.at[1])

    pltpu.sync_copy(z_hbm, acc.at[pl.ds(tid * RPT, RPT)])
    plsc.subcore_barrier()

    def issue_gathers(b, s):
        pltpu.async_copy(a_hbm.at[pl.ds((b0 + b) * K, K)], ab.at[s], sem_g[s])
        pltpu.async_copy(pd_hbm.at[gidx.at[s, 0]], gd.at[s], sem_g[s])
        pltpu.async_copy(ps_hbm.at[gidx.at[s, 1]], gs.at[s], sem_g[s])

    def wait_gathers(s):
        pltpu.make_async_copy(a_hbm.at[pl.ds(0, K)], ab.at[s], sem_g[s]).wait()
        pltpu.make_async_copy(pd_hbm.at[gidx.at[s, 0]], gd.at[s], sem_g[s]).wait()
        pltpu.make_async_copy(ps_hbm.at[gidx.at[s, 1]], gs.at[s], sem_g[s]).wait()

    pltpu.sync_copy(ei_hbm.at[pl.ds(b0, 1)], gidx.at[pl.ds(0, 1)])
    issue_gathers(0, 0)
    pltpu.async_copy(dst_hbm.at[pl.ds(b0, 1)], sidx.at[pl.ds(0, 1)], sem_d[0])
    pltpu.async_copy(ei_hbm.at[pl.ds(b0 + 1, 1)], gidx.at[pl.ds(1, 1)],
                     sem_i[1])

    @pl.loop(0, NB // 2)
    def _pair(g):
        for s in range(2):
            b = 2 * g + s
            ns = 1 - s

            @pl.when(b + 1 < NB)
            def _():
                pltpu.make_async_copy(ei_hbm.at[pl.ds(0, 1)],
                                      gidx.at[pl.ds(ns, 1)], sem_i[ns]).wait()
                issue_gathers(b + 1, ns)

                @pl.when(b >= 1)
                def _():
                    pltpu.make_async_copy(hb.at[ns], acc.at[sidx.at[ns]],
                                          sem_s[ns]).wait()
                pltpu.async_copy(dst_hbm.at[pl.ds(b0 + b + 1, 1)],
                                 sidx.at[pl.ds(ns, 1)], sem_d[ns])

            wait_gathers(s)

            @pl.when(b + 2 < NB)
            def _():
                pltpu.async_copy(ei_hbm.at[pl.ds(b0 + b + 2, 1)],
                                 gidx.at[pl.ds(s, 1)], sem_i[s])

            @plsc.parallel_loop(0, K, unroll=4)
            def _edge(e):
                for j in range(D // LN):
                    sl = pl.ds(j * LN, LN)
                    t = gd[s, e, sl] + gs[s, e, sl] + ab[s, e, sl]
                    neg = jnp.exp(t) - 1.0
                    hb[s, e, sl] = jnp.where(t > 0.0, t, neg)

            pltpu.make_async_copy(dst_hbm.at[pl.ds(0, 1)],
                                  sidx.at[pl.ds(s, 1)], sem_d[s]).wait()
            pltpu.async_copy(hb.at[s], acc.at[sidx.at[s]], sem_s[s], add=True)

    for s in range(2):
        pltpu.make_async_copy(hb.at[s], acc.at[sidx.at[s]], sem_s[s]).wait()

    plsc.subcore_barrier()
    row0 = tid * RPT
    pltpu.sync_copy(acc.at[pl.ds(row0, RPT)], out_hbm.at[cid, pl.ds(row0, RPT)])


def kernel(x, edge_index, edge_attr, phi_W1, phi_b1, phi_W2, phi_b2,
           gam_W1, gam_b1, gam_W2, gam_b2):
    w1i, w1j, w1e = phi_W1[:D], phi_W1[D:2 * D], phi_W1[2 * D:]
    g1x, g1a = gam_W1[:D], gam_W1[D:]
    src2d = edge_index[0].reshape(E // K, K)
    dst2d = edge_index[1].reshape(E // K, K)
    ei2 = jnp.stack([dst2d, src2d], axis=1)
    zeros = jnp.zeros((RPT, PAD), jnp.float32)
    hinit = jnp.zeros((K, PAD), jnp.float32).at[:, D].set(1.0)

    pd, ps = pl.pallas_call(
        _proj_body,
        grid=(10,),
        in_specs=[pl.BlockSpec((N // 10, D), lambda i: (i, 0)),
                  pl.BlockSpec((D, D), lambda i: (0, 0)),
                  pl.BlockSpec((D, D), lambda i: (0, 0))],
        out_specs=[pl.BlockSpec((N // 10, D), lambda i: (i, 0))] * 2,
        out_shape=[jax.ShapeDtypeStruct((N, D), jnp.float32)] * 2,
    )(x, w1i, w1j)

    BA = 4000
    abf = pl.pallas_call(
        _edge_affine_body,
        grid=(E // BA,),
        in_specs=[pl.BlockSpec((BA, 4), lambda i: (i, 0)),
                  pl.BlockSpec((4, D), lambda i: (0, 0)),
                  pl.BlockSpec((1, D), lambda i: (0, 0))],
        out_specs=pl.BlockSpec((BA, D), lambda i: (i, 0)),
        out_shape=jax.ShapeDtypeStruct((E, D), jnp.float32),
    )(edge_attr, w1e, phi_b1.reshape(1, D))

    hs2 = _sc_accumulate(pd, ps, abf, ei2, dst2d, zeros, hinit)

    BF = 1000
    out = pl.pallas_call(
        _final_body,
        grid=(N // BF,),
        in_specs=[pl.BlockSpec((NC, BF, PAD), lambda i: (0, i, 0)),
                  pl.BlockSpec((BF, D), lambda i: (i, 0)),
                  pl.BlockSpec((D, D), lambda i: (0, 0)),
                  pl.BlockSpec((1, D), lambda i: (0, 0)),
                  pl.BlockSpec((D, D), lambda i: (0, 0)),
                  pl.BlockSpec((D, D), lambda i: (0, 0)),
                  pl.BlockSpec((1, D), lambda i: (0, 0)),
                  pl.BlockSpec((D, D), lambda i: (0, 0)),
                  pl.BlockSpec((1, D), lambda i: (0, 0))],
        out_specs=pl.BlockSpec((BF, D), lambda i: (i, 0)),
        out_shape=jax.ShapeDtypeStruct((N, D), jnp.float32),
    )(hs2, x, phi_W2, phi_b2.reshape(1, D), g1x, g1a,
      gam_b1.reshape(1, D), gam_W2, gam_b2.reshape(1, D))

    return out
